# baseline (device time: 34755 ns/iter reference)
import jax
import jax.numpy as jnp
from jax import lax
from jax.experimental import pallas as pl
from jax.experimental.pallas import tpu as pltpu

N_DEV = 4
B, Sq, D = 2, 256, 768
Hq, Dh = 8, 64
Dq = Hq * Dh
BH = B * Hq
H2 = BH // 2
SCALE = 0.125


def kernel(x, Wq, Wo, K_ext, V_ext):
    Skv = K_ext.shape[1]

    K_h = jnp.transpose(K_ext, (0, 2, 1, 3)).astype(jnp.bfloat16)
    K_h = K_h.reshape(BH, Skv, Dh)
    V_h = jnp.transpose(V_ext, (0, 2, 1, 3)).astype(jnp.bfloat16)
    V_h = V_h.reshape(BH, Skv, Dh)

    def body(x_ref, wq_ref, wo_ref, k_ref, v_ref, out_ref,
             q_scr, q_hm, attn_scr, o_comm, s_comm,
             o_send, o_recv, s_send, s_recv):
        my = lax.axis_index("i")
        left = lax.rem(my + N_DEV - 1, N_DEV)
        right = lax.rem(my + 1, N_DEV)

        barrier = pltpu.get_barrier_semaphore()
        for nbr in (left, right):
            pl.semaphore_signal(barrier, inc=1, device_id=(nbr,),
                                device_id_type=pl.DeviceIdType.MESH)
        pl.semaphore_wait(barrier, 2)

        def rdma(src, dst, sem_i, dev, stats=False):
            send, recv = (s_send, s_recv) if stats else (o_send, o_recv)
            return pltpu.make_async_remote_copy(
                src_ref=src, dst_ref=dst,
                send_sem=send.at[sem_i], recv_sem=recv.at[sem_i],
                device_id=(dev,), device_id_type=pl.DeviceIdType.MESH)

        def mk_half(half):
            lo = half * H2
            hi = lo + H2
            mid = lo + H2 // 2
            i0 = half * 4
            return dict(
                o1r=rdma(o_comm.at[0, lo:hi], o_comm.at[1, lo:hi],
                         i0 + 0, right),
                o1l=rdma(o_comm.at[0, lo:hi], o_comm.at[2, lo:hi],
                         i0 + 1, left),
                s1r=rdma(s_comm.at[0, :, lo:hi], s_comm.at[1, :, lo:hi],
                         i0 + 0, right, stats=True),
                s1l=rdma(s_comm.at[0, :, lo:hi], s_comm.at[2, :, lo:hi],
                         i0 + 1, left, stats=True),
                o2r=rdma(o_comm.at[1, lo:mid], o_comm.at[3, lo:mid],
                         i0 + 2, right),
                o2l=rdma(o_comm.at[2, mid:hi], o_comm.at[3, mid:hi],
                         i0 + 3, left),
                s2r=rdma(s_comm.at[1, :, lo:hi], s_comm.at[3, :, lo:hi],
                         i0 + 2, right, stats=True),
            )

        ha = mk_half(0)
        hb = mk_half(1)

        xx = x_ref[...].reshape(B * Sq, D).astype(jnp.bfloat16)
        wq = wq_ref[...].astype(jnp.bfloat16)
        q_scr[...] = lax.dot(
            xx, wq, preferred_element_type=jnp.float32).astype(jnp.bfloat16)

        def partial(half):
            lo = half * H2
            hi = lo + H2
            for h in range(Hq):
                q_hm[lo + h] = (
                    q_scr[half * Sq:(half + 1) * Sq, h * Dh:(h + 1) * Dh])
            s_all = lax.dot_general(
                q_hm[lo:hi], k_ref[lo:hi], (((2,), (2,)), ((0,), (0,))),
                preferred_element_type=jnp.float32) * SCALE
            m = jnp.max(s_all, axis=2)
            p = jnp.exp(s_all - m[:, :, None])
            l = jnp.sum(p, axis=2)
            o = lax.dot_general(
                p.astype(jnp.bfloat16), v_ref[lo:hi],
                (((2,), (1,)), ((0,), (0,))),
                preferred_element_type=jnp.float32)
            o_comm[0, lo:hi] = o.astype(jnp.bfloat16)
            s_comm[0, 0, lo:hi] = m
            s_comm[0, 1, lo:hi] = l

        partial(0)
        for k_ in ("o1r", "s1r", "o1l", "s1l"):
            ha[k_].start()
        partial(1)
        for k_ in ("o1r", "s1r", "o1l", "s1l"):
            hb[k_].start()

        for hd in (ha, hb):
            hd["o1r"].wait_recv()
            hd["s1r"].wait_recv()
            hd["o2r"].start()
            hd["s2r"].start()
            hd["o1l"].wait_recv()
            hd["s1l"].wait_recv()
            hd["o2l"].start()

        s012 = s_comm[0:3]
        ms = s012[:, 0]
        ls = s012[:, 1]
        m012 = jnp.max(ms, axis=0)
        w = jnp.exp(ms - m012[None])
        l012 = jnp.sum(ls * w, axis=0)
        o012 = jnp.sum(
            o_comm[0:3].astype(jnp.float32) * w[:, :, :, None], axis=0)

        for hd in (ha, hb):
            hd["o2r"].wait_recv()
            hd["o2l"].wait_recv()
            hd["s2r"].wait_recv()
        m3 = s_comm[3, 0]
        l3 = s_comm[3, 1]
        m_star = jnp.maximum(m012, m3)
        wr = jnp.exp(m012 - m_star)
        w3 = jnp.exp(m3 - m_star)
        l_tot = l012 * wr + l3 * w3
        o_tot = (o012 * wr[:, :, None]
                 + o_comm[3].astype(jnp.float32) * w3[:, :, None])
        o_n = o_tot / l_tot[:, :, None]

        for b in range(B):
            for h in range(Hq):
                attn_scr[b * Sq:(b + 1) * Sq, h * Dh:(h + 1) * Dh] = (
                    o_n[b * Hq + h].astype(jnp.bfloat16))
        wo = wo_ref[...].astype(jnp.bfloat16)
        out_ref[...] = lax.dot(
            attn_scr[...], wo,
            preferred_element_type=jnp.float32).reshape(B, Sq, D)

        for hd in (ha, hb):
            for k_ in ("o1r", "o1l", "o2r", "o2l", "s1r", "s1l", "s2r"):
                hd[k_].wait_send()

    return pl.pallas_call(
        body,
        out_shape=jax.ShapeDtypeStruct((B, Sq, D), jnp.float32),
        in_specs=[pl.BlockSpec(memory_space=pltpu.VMEM)] * 5,
        out_specs=pl.BlockSpec(memory_space=pltpu.VMEM),
        scratch_shapes=[
            pltpu.VMEM((B * Sq, Dq), jnp.bfloat16),
            pltpu.VMEM((BH, Sq, Dh), jnp.bfloat16),
            pltpu.VMEM((B * Sq, Dq), jnp.bfloat16),
            pltpu.VMEM((N_DEV, BH, Sq, Dh), jnp.bfloat16),
            pltpu.VMEM((N_DEV, 2, BH, Sq), jnp.float32),
            pltpu.SemaphoreType.DMA((8,)),
            pltpu.SemaphoreType.DMA((8,)),
            pltpu.SemaphoreType.DMA((8,)),
            pltpu.SemaphoreType.DMA((8,)),
        ],
        compiler_params=pltpu.CompilerParams(collective_id=0),
    )(x, Wq, Wo, K_h, V_h)


# device time: 31924 ns/iter; 1.0887x vs baseline; 1.0887x over previous
import jax
import jax.numpy as jnp
from jax import lax
from jax.experimental import pallas as pl
from jax.experimental.pallas import tpu as pltpu

N_DEV = 4
B, Sq, D = 2, 256, 768
Hq, Dh = 8, 64
Dq = Hq * Dh
BH = B * Hq
H2 = BH // 2
SCALE = 0.125


def kernel(x, Wq, Wo, K_ext, V_ext):
    Skv = K_ext.shape[1]

    K_h = jnp.transpose(K_ext, (0, 2, 1, 3)).astype(jnp.bfloat16)
    K_h = K_h.reshape(BH, Skv, Dh)
    V_h = jnp.transpose(V_ext, (0, 2, 1, 3)).astype(jnp.bfloat16)
    V_h = V_h.reshape(BH, Skv, Dh)

    def body(x_ref, wq_ref, wo_ref, k_ref, v_ref, out_ref,
             q_scr, q_hm, attn_scr, o_comm, s_comm,
             o_send, o_recv, s_send, s_recv):
        my = lax.axis_index("i")
        left = lax.rem(my + N_DEV - 1, N_DEV)
        right = lax.rem(my + 1, N_DEV)

        barrier = pltpu.get_barrier_semaphore()
        for nbr in (left, right):
            pl.semaphore_signal(barrier, inc=1, device_id=(nbr,),
                                device_id_type=pl.DeviceIdType.MESH)
        pl.semaphore_wait(barrier, 2)

        def rdma(src, dst, sem_i, dev, stats=False):
            send, recv = (s_send, s_recv) if stats else (o_send, o_recv)
            return pltpu.make_async_remote_copy(
                src_ref=src, dst_ref=dst,
                send_sem=send.at[sem_i], recv_sem=recv.at[sem_i],
                device_id=(dev,), device_id_type=pl.DeviceIdType.MESH)

        def mk_half(half):
            lo = half * H2
            hi = lo + H2
            mid = lo + H2 // 2
            i0 = half * 4
            return dict(
                o1r=rdma(o_comm.at[0, lo:hi], o_comm.at[1, lo:hi],
                         i0 + 0, right),
                o1l=rdma(o_comm.at[0, lo:hi], o_comm.at[2, lo:hi],
                         i0 + 1, left),
                s1r=rdma(s_comm.at[0, :, lo:hi], s_comm.at[1, :, lo:hi],
                         i0 + 0, right, stats=True),
                s1l=rdma(s_comm.at[0, :, lo:hi], s_comm.at[2, :, lo:hi],
                         i0 + 1, left, stats=True),
                o2r=rdma(o_comm.at[1, lo:mid], o_comm.at[3, lo:mid],
                         i0 + 2, right),
                o2l=rdma(o_comm.at[2, mid:hi], o_comm.at[3, mid:hi],
                         i0 + 3, left),
                s2r=rdma(s_comm.at[1, :, lo:hi], s_comm.at[3, :, lo:hi],
                         i0 + 2, right, stats=True),
            )

        ha = mk_half(0)
        hb = mk_half(1)

        xx = x_ref[...].reshape(B * Sq, D).astype(jnp.bfloat16)
        wq = wq_ref[...].astype(jnp.bfloat16)
        q_scr[...] = (lax.dot(
            xx, wq, preferred_element_type=jnp.float32)
            * SCALE).astype(jnp.bfloat16)

        def partial(half):
            lo = half * H2
            hi = lo + H2
            for h in range(Hq):
                q_hm[lo + h] = (
                    q_scr[half * Sq:(half + 1) * Sq, h * Dh:(h + 1) * Dh])
            s_all = lax.dot_general(
                q_hm[lo:hi], k_ref[lo:hi], (((2,), (2,)), ((0,), (0,))),
                preferred_element_type=jnp.float32)
            m = jnp.max(s_all, axis=2)
            p = jnp.exp(s_all - m[:, :, None])
            l = jnp.sum(p, axis=2)
            o = lax.dot_general(
                p.astype(jnp.bfloat16), v_ref[lo:hi],
                (((2,), (1,)), ((0,), (0,))),
                preferred_element_type=jnp.float32)
            o_comm[0, lo:hi] = o.astype(jnp.bfloat16)
            s_comm[0, 0, lo:hi] = m
            s_comm[0, 1, lo:hi] = l

        partial(0)
        for k_ in ("o1r", "s1r", "o1l", "s1l"):
            ha[k_].start()
        partial(1)
        for k_ in ("o1r", "s1r", "o1l", "s1l"):
            hb[k_].start()

        wo = wo_ref[...].astype(jnp.bfloat16)

        merged = {}
        for half, hd in ((0, ha), (1, hb)):
            lo = half * H2
            hi = lo + H2
            hd["o1r"].wait_recv()
            hd["s1r"].wait_recv()
            hd["o2r"].start()
            hd["s2r"].start()
            hd["o1l"].wait_recv()
            hd["s1l"].wait_recv()
            hd["o2l"].start()
            s012 = s_comm[0:3, :, lo:hi]
            ms = s012[:, 0]
            ls = s012[:, 1]
            m012 = jnp.max(ms, axis=0)
            w = jnp.exp(ms - m012[None])
            l012 = jnp.sum(ls * w, axis=0)
            o012 = jnp.sum(
                o_comm[0:3, lo:hi].astype(jnp.float32) * w[:, :, :, None],
                axis=0)
            merged[half] = (m012, l012, o012)

        for half, hd in ((0, ha), (1, hb)):
            lo = half * H2
            hi = lo + H2
            hd["o2r"].wait_recv()
            hd["o2l"].wait_recv()
            hd["s2r"].wait_recv()
            m012, l012, o012 = merged[half]
            m3 = s_comm[3, 0, lo:hi]
            l3 = s_comm[3, 1, lo:hi]
            m_star = jnp.maximum(m012, m3)
            wr = jnp.exp(m012 - m_star)
            w3 = jnp.exp(m3 - m_star)
            l_tot = l012 * wr + l3 * w3
            wrn = wr / l_tot
            w3n = w3 / l_tot
            o_n = (o012 * wrn[:, :, None]
                   + o_comm[3, lo:hi].astype(jnp.float32) * w3n[:, :, None])
            for h in range(Hq):
                attn_scr[half * Sq:(half + 1) * Sq, h * Dh:(h + 1) * Dh] = (
                    o_n[h].astype(jnp.bfloat16))
            out_ref[half] = lax.dot(
                attn_scr[half * Sq:(half + 1) * Sq], wo,
                preferred_element_type=jnp.float32)

        for hd in (ha, hb):
            for k_ in ("o1r", "o1l", "o2r", "o2l", "s1r", "s1l", "s2r"):
                hd[k_].wait_send()

    return pl.pallas_call(
        body,
        out_shape=jax.ShapeDtypeStruct((B, Sq, D), jnp.float32),
        in_specs=[pl.BlockSpec(memory_space=pltpu.VMEM)] * 5,
        out_specs=pl.BlockSpec(memory_space=pltpu.VMEM),
        scratch_shapes=[
            pltpu.VMEM((B * Sq, Dq), jnp.bfloat16),
            pltpu.VMEM((BH, Sq, Dh), jnp.bfloat16),
            pltpu.VMEM((B * Sq, Dq), jnp.bfloat16),
            pltpu.VMEM((N_DEV, BH, Sq, Dh), jnp.bfloat16),
            pltpu.VMEM((N_DEV, 2, BH, Sq), jnp.float32),
            pltpu.SemaphoreType.DMA((8,)),
            pltpu.SemaphoreType.DMA((8,)),
            pltpu.SemaphoreType.DMA((8,)),
            pltpu.SemaphoreType.DMA((8,)),
        ],
        compiler_params=pltpu.CompilerParams(collective_id=0),
    )(x, Wq, Wo, K_h, V_h)
